# Initial kernel scaffold; baseline (speedup 1.0000x reference)
import jax
import jax.numpy as jnp
from jax.experimental import pallas as pl


def kernel(xyz1, xyz2, points1, points2, W1, b1, g1, be1, W2, b2, g2, be2):
    raise NotImplementedError("write your pallas kernel here")


# speedup vs baseline: 8.1986x; 8.1986x over previous
import functools

import jax
import jax.numpy as jnp
from jax import lax
from jax.experimental import pallas as pl
from jax.experimental.pallas import tpu as pltpu
from jax.experimental.pallas import tpu_sc as plsc

_TN = 512


def _knn_body(x1_ref, x2_ref, i0_ref, i1_ref, i2_ref, w0_ref, w1_ref, w2_ref):
    b = pl.program_id(0)
    x1 = x1_ref[0]
    x2 = x2_ref[0]
    s = x2.shape[1]
    d = lax.dot_general(x1, x2, (((0,), (0,)), ((), ())),
                        preferred_element_type=jnp.float32)
    d = (-2.0) * d
    d = d + jnp.sum(x1 * x1, axis=0)[:, None]
    d = d + jnp.sum(x2 * x2, axis=0)[None, :]
    iota = lax.broadcasted_iota(jnp.int32, d.shape, 1)
    idxs = []
    vals = []
    for k in range(3):
        m = jnp.min(d, axis=1)
        cand = jnp.where(d == m[:, None], iota, s)
        ik = jnp.min(cand, axis=1)
        idxs.append(ik)
        vals.append(m)
        if k < 2:
            d = jnp.where(iota == ik[:, None], jnp.float32(jnp.inf), d)
    r = [1.0 / (v + 1e-8) for v in vals]
    norm = r[0] + r[1] + r[2]
    off = b * s
    for k, (iref, wref) in enumerate(
            zip((i0_ref, i1_ref, i2_ref), (w0_ref, w1_ref, w2_ref))):
        iref[0, 0, :] = idxs[k] + off
        wref[0, 0, :] = r[k] / norm


def _knn(xyz1, xyz2):
    B, _, N = xyz1.shape
    S = xyz2.shape[2]
    grid = (B, N // _TN)
    ispec = jax.ShapeDtypeStruct((B, 1, N), jnp.int32)
    wspec = jax.ShapeDtypeStruct((B, 1, N), jnp.float32)
    out_specs = [pl.BlockSpec((1, 1, _TN), lambda b, n: (b, 0, n))] * 6
    return pl.pallas_call(
        _knn_body,
        grid=grid,
        in_specs=[
            pl.BlockSpec((1, 3, _TN), lambda b, n: (b, 0, n)),
            pl.BlockSpec((1, 3, S), lambda b, n: (b, 0, 0)),
        ],
        out_specs=out_specs,
        out_shape=[ispec, ispec, ispec, wspec, wspec, wspec],
    )(xyz1, xyz2)


_QC = 128


def _interp(table, idx0, idx1, idx2, wgt0, wgt1, wgt2):
    BN = idx0.shape[0]
    D = table.shape[1]
    L = 16
    NC, NS = 2, 16
    NW = NC * NS
    QT = BN // NW
    mesh = plsc.VectorSubcoreMesh(core_axis_name="c", subcore_axis_name="s")

    @functools.partial(
        pl.kernel,
        mesh=mesh,
        out_type=jax.ShapeDtypeStruct((BN, D), jnp.float32),
        scratch_types=[
            pltpu.VMEM((_QC,), jnp.int32),
            pltpu.VMEM((_QC,), jnp.int32),
            pltpu.VMEM((_QC,), jnp.int32),
            pltpu.VMEM((_QC, D), jnp.float32),
            pltpu.VMEM((_QC, D), jnp.float32),
            pltpu.VMEM((_QC, D), jnp.float32),
            pltpu.VMEM((_QC, L), jnp.float32),
            pltpu.VMEM((_QC, L), jnp.float32),
            pltpu.VMEM((_QC, L), jnp.float32),
            pltpu.VMEM((_QC, D), jnp.float32),
            pltpu.SemaphoreType.DMA,
        ],
    )
    def body(table_hbm, i0_hbm, i1_hbm, i2_hbm, w0_hbm, w1_hbm, w2_hbm,
             out_hbm, i0_v, i1_v, i2_v, r0_v, r1_v, r2_v, w0_v, w1_v, w2_v,
             out_v, sem):
        wid = lax.axis_index("s") * NC + lax.axis_index("c")
        base = wid * QT
        idx_refs = (i0_v, i1_v, i2_v)
        row_refs = (r0_v, r1_v, r2_v)
        wgt_refs = (w0_v, w1_v, w2_v)

        def chunk(j, carry):
            qb = base + j * _QC
            for k, (ih, wh) in enumerate(
                    zip((i0_hbm, i1_hbm, i2_hbm), (w0_hbm, w1_hbm, w2_hbm))):
                pltpu.sync_copy(ih.at[pl.ds(qb, _QC)], idx_refs[k])
                pltpu.sync_copy(wh.at[pl.ds(qb, _QC), :], wgt_refs[k])
            copies = [
                pltpu.async_copy(table_hbm.at[idx_refs[k]], row_refs[k], sem)
                for k in range(3)
            ]
            for c in copies:
                c.wait()

            def one_q(q, c2):
                for c in range(D // L):
                    sl = pl.ds(c * L, L)
                    acc = w0_v[q, :] * r0_v[q, sl]
                    acc = acc + w1_v[q, :] * r1_v[q, sl]
                    acc = acc + w2_v[q, :] * r2_v[q, sl]
                    out_v[q, sl] = acc
                return c2

            lax.fori_loop(0, _QC, one_q, 0)
            pltpu.sync_copy(out_v, out_hbm.at[pl.ds(qb, _QC), :])
            return carry

        lax.fori_loop(0, QT // _QC, chunk, 0)

    return body(table, idx0, idx1, idx2, wgt0, wgt1, wgt2)


_RB = 128


def _mlp1_body(x_ref, w_ref, b_ref, g_ref, be_ref, o_ref):
    y = lax.dot_general(w_ref[...], x_ref[...], (((1,), (0,)), ((), ())),
                        preferred_element_type=jnp.float32)
    y = y + b_ref[...]
    m = jnp.mean(y, axis=1, keepdims=True)
    yc = y - m
    v = jnp.mean(yc * yc, axis=1, keepdims=True)
    yh = yc * lax.rsqrt(v + 1e-5)
    o_ref[...] = jnp.maximum(g_ref[...] * yh + be_ref[...], 0.0)


def _mlp1(X, W1, b1, g1, be1):
    O, C = W1.shape
    Ncol = X.shape[1]
    return pl.pallas_call(
        _mlp1_body,
        grid=(O // _RB,),
        in_specs=[
            pl.BlockSpec((C, Ncol), lambda i: (0, 0)),
            pl.BlockSpec((_RB, C), lambda i: (i, 0)),
            pl.BlockSpec((_RB, 1), lambda i: (i, 0)),
            pl.BlockSpec((_RB, 1), lambda i: (i, 0)),
            pl.BlockSpec((_RB, 1), lambda i: (i, 0)),
        ],
        out_specs=pl.BlockSpec((_RB, Ncol), lambda i: (i, 0)),
        out_shape=jax.ShapeDtypeStruct((O, Ncol), jnp.float32),
    )(X, W1, b1[:, None], g1[:, None], be1[:, None])


def _mlp2_body(x_ref, w_ref, b_ref, g_ref, be_ref, o_ref):
    y = lax.dot_general(w_ref[...], x_ref[...], (((1,), (0,)), ((), ())),
                        preferred_element_type=jnp.float32)
    y = y + b_ref[...]
    m = jnp.mean(y, axis=1, keepdims=True)
    yc = y - m
    v = jnp.mean(yc * yc, axis=1, keepdims=True)
    yh = yc * lax.rsqrt(v + 1e-5)
    o_ref[...] = jnp.maximum(g_ref[...] * yh + be_ref[...], 0.0)


def _mlp2(Y1, W2, b2, g2, be2):
    O, C = W2.shape
    Ncol = Y1.shape[1]
    return pl.pallas_call(
        _mlp2_body,
        out_shape=jax.ShapeDtypeStruct((O, Ncol), jnp.float32),
    )(Y1, W2, b2[:, None], g2[:, None], be2[:, None])


def kernel(xyz1, xyz2, points1, points2, W1, b1, g1, be1, W2, b2, g2, be2):
    B, _, N = xyz1.shape
    S = xyz2.shape[2]
    D = points2.shape[1]
    BN = B * N

    i0, i1, i2, w0, w1, w2 = _knn(xyz1, xyz2)
    idxs = [i.reshape(BN) for i in (i0, i1, i2)]
    wgts = [jnp.broadcast_to(w.reshape(BN)[:, None], (BN, 16))
            for w in (w0, w1, w2)]

    table = points2.transpose(0, 2, 1).reshape(B * S, D)
    interp = _interp(table, *idxs, *wgts)

    X1 = points1.transpose(2, 0, 1).reshape(N, B * D)
    X2 = interp.reshape(B, N, D).transpose(1, 0, 2).reshape(N, B * D)
    X = jnp.concatenate([X1, X2], axis=0)

    Y1 = _mlp1(X, W1, b1, g1, be1)
    Y2 = _mlp2(Y1, W2, b2, g2, be2)
    return Y2.reshape(W2.shape[0], B, D).transpose(1, 0, 2)
